# Initial kernel scaffold; baseline (speedup 1.0000x reference)
import jax
import jax.numpy as jnp
from jax.experimental import pallas as pl


def kernel(x, edge_index, W1, b1, W2, b2, W3, b3):
    raise NotImplementedError("write your pallas kernel here")


# speedup vs baseline: 6.4930x; 6.4930x over previous
import functools

import jax
import jax.numpy as jnp
from jax import lax
from jax.experimental import pallas as pl
from jax.experimental.pallas import tpu as pltpu
from jax.experimental.pallas import tpu_sc as plsc

NC = 2
NS = 16
NW = NC * NS


def _prep_call(edge_index, n, e):
    eb = 32000

    def k(e_ref, o_ref):
        r = e_ref[0, :]
        c = e_ref[1, :]
        self_m = r == c
        o_ref[0, :] = jnp.where(self_m, n, r)
        o_ref[1, :] = jnp.where(self_m, n, c)

    return pl.pallas_call(
        k,
        grid=(e // eb,),
        in_specs=[pl.BlockSpec((2, eb), lambda i: (0, i))],
        out_specs=pl.BlockSpec((2, eb), lambda i: (0, i)),
        out_shape=jax.ShapeDtypeStruct((2, e), jnp.int32),
    )(edge_index)


def _deg_call(rowp, zeros1, np_, e):
    epw = e // NW
    ch = 80
    nwin = epw // ch
    rpt = np_ // NS
    mesh = plsc.VectorSubcoreMesh(core_axis_name="c", subcore_axis_name="s")

    @functools.partial(
        pl.kernel,
        out_type=jax.ShapeDtypeStruct((NC, np_), jnp.float32),
        mesh=mesh,
        scratch_types=[
            pltpu.VMEM((ch,), jnp.int32),
            pltpu.VMEM((ch,), jnp.float32),
            pltpu.VMEM_SHARED((np_,), jnp.float32),
        ],
    )
    def k(rowp_hbm, z_hbm, out_hbm, sidx, ones, acc):
        c = lax.axis_index("c")
        s = lax.axis_index("s")
        wid = c * NS + s
        for j in range(ch // 16):
            ones[pl.ds(j * 16, 16)] = jnp.ones((16,), jnp.float32)
        pltpu.sync_copy(z_hbm.at[pl.ds(s * rpt, rpt)], acc.at[pl.ds(s * rpt, rpt)])
        plsc.subcore_barrier()
        base = wid * epw

        def body(w, carry):
            off = pl.multiple_of(base + w * ch, 8)
            pltpu.sync_copy(rowp_hbm.at[pl.ds(off, ch)], sidx)
            pltpu.sync_copy(ones, acc.at[sidx], add=True)
            return carry

        lax.fori_loop(0, nwin, body, 0)
        plsc.subcore_barrier()
        pltpu.sync_copy(acc.at[pl.ds(s * rpt, rpt)],
                        out_hbm.at[c, pl.ds(s * rpt, rpt)])

    return k(rowp, zeros1)


def _sc_prop_call(u, row, colp, zeros2, np_, d, e):
    epw = e // NW
    ch = 80
    nwin = epw // ch
    rpt = np_ // NS
    mesh = plsc.VectorSubcoreMesh(core_axis_name="c", subcore_axis_name="s")

    @functools.partial(
        pl.kernel,
        out_type=jax.ShapeDtypeStruct((NC, np_, d), jnp.float32),
        mesh=mesh,
        scratch_types=[
            pltpu.VMEM((ch,), jnp.int32),
            pltpu.VMEM((ch,), jnp.int32),
            pltpu.VMEM((ch, d), jnp.float32),
            pltpu.VMEM_SHARED((np_, d), jnp.float32),
            pltpu.SemaphoreType.DMA,
        ],
    )
    def k(u_hbm, row_hbm, colp_hbm, z_hbm, out_hbm, gidx, sidx, rows, acc, sem):
        c = lax.axis_index("c")
        s = lax.axis_index("s")
        wid = c * NS + s
        pltpu.sync_copy(z_hbm.at[pl.ds(s * rpt, rpt)], acc.at[pl.ds(s * rpt, rpt)])
        plsc.subcore_barrier()
        base = wid * epw

        def body(w, carry):
            off = pl.multiple_of(base + w * ch, 8)
            pltpu.sync_copy(row_hbm.at[pl.ds(off, ch)], gidx)
            pltpu.async_copy(u_hbm.at[gidx], rows, sem).wait()
            pltpu.sync_copy(colp_hbm.at[pl.ds(off, ch)], sidx)
            pltpu.sync_copy(rows, acc.at[sidx], add=True)
            return carry

        lax.fori_loop(0, nwin, body, 0)
        plsc.subcore_barrier()
        pltpu.sync_copy(acc.at[pl.ds(s * rpt, rpt)],
                        out_hbm.at[c, pl.ds(s * rpt, rpt)])

    return k(u, row, colp, zeros2)


def _dis_u_call(degp, x_pad, np_, d):
    def k(degp_ref, x_ref, dis_ref, u_ref):
        dg = degp_ref[0, :] + degp_ref[1, :]
        dis = jnp.where(dg > 0, lax.rsqrt(dg), 0.0)
        dis_b = jnp.broadcast_to(dis[:, None], (np_, d))
        dis_ref[...] = dis_b
        u_ref[...] = dis_b * x_ref[...]

    return pl.pallas_call(
        k,
        out_shape=(
            jax.ShapeDtypeStruct((np_, d), jnp.float32),
            jax.ShapeDtypeStruct((np_, d), jnp.float32),
        ),
    )(degp, x_pad)


def _layer_c_call(h, sp, dis_b, w, np_, d):
    blk = 1024

    def k(h_ref, sp_ref, dis_ref, w_ref, outp_ref, v_ref):
        s1 = sp_ref[0] + sp_ref[1]
        dis = dis_ref[...]
        tx1 = -(dis * s1)
        v_ref[...] = dis * tx1
        outp_ref[...] = (
            jnp.dot(h_ref[...], w_ref[0], preferred_element_type=jnp.float32)
            + jnp.dot(tx1, w_ref[1], preferred_element_type=jnp.float32)
        )

    return pl.pallas_call(
        k,
        grid=(np_ // blk,),
        in_specs=[
            pl.BlockSpec((blk, d), lambda i: (i, 0)),
            pl.BlockSpec((NC, blk, d), lambda i: (0, i, 0)),
            pl.BlockSpec((blk, d), lambda i: (i, 0)),
            pl.BlockSpec((3, d, d), lambda i: (0, 0, 0)),
        ],
        out_specs=(
            pl.BlockSpec((blk, d), lambda i: (i, 0)),
            pl.BlockSpec((blk, d), lambda i: (i, 0)),
        ),
        out_shape=(
            jax.ShapeDtypeStruct((np_, d), jnp.float32),
            jax.ShapeDtypeStruct((np_, d), jnp.float32),
        ),
    )(h, sp, dis_b, w)


def _layer_d_call(h, outp, sp, dis_b, w, b, np_, d, relu):
    blk = 1024

    def k(h_ref, outp_ref, sp_ref, dis_ref, w_ref, b_ref, o_ref, u_ref):
        s2 = sp_ref[0] + sp_ref[1]
        dis = dis_ref[...]
        tx2 = -2.0 * (dis * s2) - h_ref[...]
        o = (
            outp_ref[...]
            + jnp.dot(tx2, w_ref[2], preferred_element_type=jnp.float32)
            + b_ref[...]
        )
        if relu:
            o = jnp.maximum(o, 0.0)
        o_ref[...] = o
        u_ref[...] = dis * o

    return pl.pallas_call(
        k,
        grid=(np_ // blk,),
        in_specs=[
            pl.BlockSpec((blk, d), lambda i: (i, 0)),
            pl.BlockSpec((blk, d), lambda i: (i, 0)),
            pl.BlockSpec((NC, blk, d), lambda i: (0, i, 0)),
            pl.BlockSpec((blk, d), lambda i: (i, 0)),
            pl.BlockSpec((3, d, d), lambda i: (0, 0, 0)),
            pl.BlockSpec((1, d), lambda i: (0, 0)),
        ],
        out_specs=(
            pl.BlockSpec((blk, d), lambda i: (i, 0)),
            pl.BlockSpec((blk, d), lambda i: (i, 0)),
        ),
        out_shape=(
            jax.ShapeDtypeStruct((np_, d), jnp.float32),
            jax.ShapeDtypeStruct((np_, d), jnp.float32),
        ),
    )(h, outp, sp, dis_b, w, b)


def kernel(x, edge_index, W1, b1, W2, b2, W3, b3):
    n, d = x.shape
    e = edge_index.shape[1]
    np_ = ((n + 1 + NW * 8 - 1) // (NW * 8)) * (NW * 8)

    rc = _prep_call(edge_index, n, e)
    rowp = rc[0]
    colp = rc[1]
    row = edge_index[0]

    zeros1 = jnp.zeros((np_,), jnp.float32)
    zeros2 = jnp.zeros((np_, d), jnp.float32)
    x_pad = jnp.pad(x, ((0, np_ - n), (0, 0)))

    degp = _deg_call(rowp, zeros1, np_, e)
    dis_b, u = _dis_u_call(degp, x_pad, np_, d)

    h = x_pad
    for w, b, relu in ((W1, b1, True), (W2, b2, True), (W3, b3, False)):
        b2d = b.reshape(1, d)
        sp1 = _sc_prop_call(u, row, colp, zeros2, np_, d, e)
        outp, v = _layer_c_call(h, sp1, dis_b, w, np_, d)
        sp2 = _sc_prop_call(v, row, colp, zeros2, np_, d, e)
        h, u = _layer_d_call(h, outp, sp2, dis_b, w, b2d, np_, d, relu)

    return h[:n]


# speedup vs baseline: 15.3763x; 2.3681x over previous
import functools

import jax
import jax.numpy as jnp
from jax import lax
from jax.experimental import pallas as pl
from jax.experimental.pallas import tpu as pltpu
from jax.experimental.pallas import tpu_sc as plsc

NC = 2
NS = 16
NW = NC * NS


def _prep_call(edge_index, n, e):
    eb = 32000

    def k(e_ref, o_ref):
        r = e_ref[0, :]
        c = e_ref[1, :]
        self_m = r == c
        o_ref[0, :] = jnp.where(self_m, n, r)
        o_ref[1, :] = jnp.where(self_m, n, c)

    return pl.pallas_call(
        k,
        grid=(e // eb,),
        in_specs=[pl.BlockSpec((2, eb), lambda i: (0, i))],
        out_specs=pl.BlockSpec((2, eb), lambda i: (0, i)),
        out_shape=jax.ShapeDtypeStruct((2, e), jnp.int32),
    )(edge_index)


def _deg_call(rowp, zeros1, np_, e):
    epw = e // NW
    ch = 80
    nwin = epw // ch
    rpt = np_ // NS
    mesh = plsc.VectorSubcoreMesh(core_axis_name="c", subcore_axis_name="s")

    @functools.partial(
        pl.kernel,
        out_type=jax.ShapeDtypeStruct((NC, np_), jnp.float32),
        mesh=mesh,
        scratch_types=[
            pltpu.VMEM((nwin, ch), jnp.int32),
            pltpu.VMEM((ch,), jnp.float32),
            pltpu.VMEM_SHARED((np_,), jnp.float32),
        ],
    )
    def k(rowp_hbm, z_hbm, out_hbm, sidx, ones, acc):
        c = lax.axis_index("c")
        s = lax.axis_index("s")
        wid = c * NS + s
        for j in range(ch // 16):
            ones[pl.ds(j * 16, 16)] = jnp.ones((16,), jnp.float32)
        pltpu.sync_copy(z_hbm.at[pl.ds(s * rpt, rpt)], acc.at[pl.ds(s * rpt, rpt)])
        pltpu.sync_copy(rowp_hbm.at[wid], sidx)
        plsc.subcore_barrier()

        def body(w, carry):
            pltpu.sync_copy(ones, acc.at[sidx.at[w]], add=True)
            return carry

        lax.fori_loop(0, nwin, body, 0)
        plsc.subcore_barrier()
        pltpu.sync_copy(acc.at[pl.ds(s * rpt, rpt)],
                        out_hbm.at[c, pl.ds(s * rpt, rpt)])

    return k(rowp, zeros1)


def _sc_prop_call(u, row, colp, zeros2, np_, d, e):
    epw = e // NW
    ch = 80
    nwin = epw // ch
    rpt = np_ // NS
    mesh = plsc.VectorSubcoreMesh(core_axis_name="c", subcore_axis_name="s")

    @functools.partial(
        pl.kernel,
        out_type=jax.ShapeDtypeStruct((NC, np_, d), jnp.float32),
        mesh=mesh,
        scratch_types=[
            pltpu.VMEM((epw,), jnp.int32),
            pltpu.VMEM((nwin, ch), jnp.int32),
            pltpu.VMEM((ch, d), jnp.float32),
            pltpu.VMEM((ch, d), jnp.float32),
            pltpu.VMEM_SHARED((np_, d), jnp.float32),
            pltpu.SemaphoreType.DMA,
            pltpu.SemaphoreType.DMA,
        ],
    )
    def k(u_hbm, row_hbm, colp_hbm, z_hbm, out_hbm,
          gidx, sidx, rows0, rows1, acc, sem0, sem1):
        c = lax.axis_index("c")
        s = lax.axis_index("s")
        wid = c * NS + s
        rows = (rows0, rows1)
        sems = (sem0, sem1)
        pltpu.sync_copy(z_hbm.at[pl.ds(s * rpt, rpt)], acc.at[pl.ds(s * rpt, rpt)])
        pltpu.sync_copy(row_hbm.at[pl.ds(wid * epw, epw)], gidx)
        pltpu.sync_copy(colp_hbm.at[wid], sidx)
        plsc.subcore_barrier()

        def issue(wi, b):
            off = pl.multiple_of(wi * ch, 8)
            pltpu.async_copy(u_hbm.at[gidx.at[pl.ds(off, ch)]], rows[b], sems[b])

        def wait(b):
            pltpu.make_async_copy(
                u_hbm.at[gidx.at[pl.ds(0, ch)]], rows[b], sems[b]).wait()

        issue(0, 0)
        issue(1, 1)

        def body(g, carry):
            for b in range(2):
                wi = g * 2 + b

                @pl.when(wi < nwin)
                def _():
                    wait(b)
                    pltpu.sync_copy(rows[b], acc.at[sidx.at[wi]], add=True)

                    @pl.when(wi + 2 < nwin)
                    def _():
                        issue(wi + 2, b)

            return carry

        lax.fori_loop(0, (nwin + 1) // 2, body, 0)
        plsc.subcore_barrier()
        pltpu.sync_copy(acc.at[pl.ds(s * rpt, rpt)],
                        out_hbm.at[c, pl.ds(s * rpt, rpt)])

    return k(u, row, colp, zeros2)


def _dis_u_call(degp, x_pad, np_, d):
    def k(degp_ref, x_ref, dis_ref, u_ref):
        dg = degp_ref[0, :] + degp_ref[1, :]
        dis = jnp.where(dg > 0, lax.rsqrt(dg), 0.0)
        dis_b = jnp.broadcast_to(dis[:, None], (np_, d))
        dis_ref[...] = dis_b
        u_ref[...] = dis_b * x_ref[...]

    return pl.pallas_call(
        k,
        out_shape=(
            jax.ShapeDtypeStruct((np_, d), jnp.float32),
            jax.ShapeDtypeStruct((np_, d), jnp.float32),
        ),
    )(degp, x_pad)


def _layer_c_call(h, sp, dis_b, w, np_, d):
    blk = 1024

    def k(h_ref, sp_ref, dis_ref, w_ref, outp_ref, v_ref):
        s1 = sp_ref[0] + sp_ref[1]
        dis = dis_ref[...]
        tx1 = -(dis * s1)
        v_ref[...] = dis * tx1
        outp_ref[...] = (
            jnp.dot(h_ref[...], w_ref[0], preferred_element_type=jnp.float32)
            + jnp.dot(tx1, w_ref[1], preferred_element_type=jnp.float32)
        )

    return pl.pallas_call(
        k,
        grid=(np_ // blk,),
        in_specs=[
            pl.BlockSpec((blk, d), lambda i: (i, 0)),
            pl.BlockSpec((NC, blk, d), lambda i: (0, i, 0)),
            pl.BlockSpec((blk, d), lambda i: (i, 0)),
            pl.BlockSpec((3, d, d), lambda i: (0, 0, 0)),
        ],
        out_specs=(
            pl.BlockSpec((blk, d), lambda i: (i, 0)),
            pl.BlockSpec((blk, d), lambda i: (i, 0)),
        ),
        out_shape=(
            jax.ShapeDtypeStruct((np_, d), jnp.float32),
            jax.ShapeDtypeStruct((np_, d), jnp.float32),
        ),
    )(h, sp, dis_b, w)


def _layer_d_call(h, outp, sp, dis_b, w, b, np_, d, relu):
    blk = 1024

    def k(h_ref, outp_ref, sp_ref, dis_ref, w_ref, b_ref, o_ref, u_ref):
        s2 = sp_ref[0] + sp_ref[1]
        dis = dis_ref[...]
        tx2 = -2.0 * (dis * s2) - h_ref[...]
        o = (
            outp_ref[...]
            + jnp.dot(tx2, w_ref[2], preferred_element_type=jnp.float32)
            + b_ref[...]
        )
        if relu:
            o = jnp.maximum(o, 0.0)
        o_ref[...] = o
        u_ref[...] = dis * o

    return pl.pallas_call(
        k,
        grid=(np_ // blk,),
        in_specs=[
            pl.BlockSpec((blk, d), lambda i: (i, 0)),
            pl.BlockSpec((blk, d), lambda i: (i, 0)),
            pl.BlockSpec((NC, blk, d), lambda i: (0, i, 0)),
            pl.BlockSpec((blk, d), lambda i: (i, 0)),
            pl.BlockSpec((3, d, d), lambda i: (0, 0, 0)),
            pl.BlockSpec((1, d), lambda i: (0, 0)),
        ],
        out_specs=(
            pl.BlockSpec((blk, d), lambda i: (i, 0)),
            pl.BlockSpec((blk, d), lambda i: (i, 0)),
        ),
        out_shape=(
            jax.ShapeDtypeStruct((np_, d), jnp.float32),
            jax.ShapeDtypeStruct((np_, d), jnp.float32),
        ),
    )(h, outp, sp, dis_b, w, b)


def kernel(x, edge_index, W1, b1, W2, b2, W3, b3):
    n, d = x.shape
    e = edge_index.shape[1]
    np_ = ((n + 1 + NW * 8 - 1) // (NW * 8)) * (NW * 8)

    epw = e // NW
    ch = 80
    nwin = epw // ch
    rc = _prep_call(edge_index, n, e)
    rowp = rc[0].reshape(NW, nwin, ch)
    colp = rc[1].reshape(NW, nwin, ch)
    row = edge_index[0]

    zeros1 = jnp.zeros((np_,), jnp.float32)
    zeros2 = jnp.zeros((np_, d), jnp.float32)
    x_pad = jnp.pad(x, ((0, np_ - n), (0, 0)))

    degp = _deg_call(rowp, zeros1, np_, e)
    dis_b, u = _dis_u_call(degp, x_pad, np_, d)

    h = x_pad
    for w, b, relu in ((W1, b1, True), (W2, b2, True), (W3, b3, False)):
        b2d = b.reshape(1, d)
        sp1 = _sc_prop_call(u, row, colp, zeros2, np_, d, e)
        outp, v = _layer_c_call(h, sp1, dis_b, w, np_, d)
        sp2 = _sc_prop_call(v, row, colp, zeros2, np_, d, e)
        h, u = _layer_d_call(h, outp, sp2, dis_b, w, b2d, np_, d, relu)

    return h[:n]


# speedup vs baseline: 17.7411x; 1.1538x over previous
import functools

import jax
import jax.numpy as jnp
from jax import lax
from jax.experimental import pallas as pl
from jax.experimental.pallas import tpu as pltpu
from jax.experimental.pallas import tpu_sc as plsc

NC = 2
NS = 16
NW = NC * NS


def _prep_call(edge_index, n, e):
    eb = 32000

    def k(e_ref, o_ref):
        r = e_ref[0, :]
        c = e_ref[1, :]
        self_m = r == c
        o_ref[0, :] = jnp.where(self_m, n, r)
        o_ref[1, :] = jnp.where(self_m, n, c)

    return pl.pallas_call(
        k,
        grid=(e // eb,),
        in_specs=[pl.BlockSpec((2, eb), lambda i: (0, i))],
        out_specs=pl.BlockSpec((2, eb), lambda i: (0, i)),
        out_shape=jax.ShapeDtypeStruct((2, e), jnp.int32),
    )(edge_index)


def _deg_call(rowp, zeros1, np_, e):
    epw = e // NW
    ch = 80
    nwin = epw // ch
    rpt = np_ // NS
    mesh = plsc.VectorSubcoreMesh(core_axis_name="c", subcore_axis_name="s")

    @functools.partial(
        pl.kernel,
        out_type=jax.ShapeDtypeStruct((NC, np_), jnp.float32),
        mesh=mesh,
        scratch_types=[
            pltpu.VMEM((nwin, ch), jnp.int32),
            pltpu.VMEM((ch,), jnp.float32),
            pltpu.VMEM_SHARED((np_,), jnp.float32),
            pltpu.SemaphoreType.DMA,
        ],
    )
    def k(rowp_hbm, z_hbm, out_hbm, sidx, ones, acc, ssem):
        c = lax.axis_index("c")
        s = lax.axis_index("s")
        wid = c * NS + s
        for j in range(ch // 16):
            ones[pl.ds(j * 16, 16)] = jnp.ones((16,), jnp.float32)
        pltpu.sync_copy(z_hbm.at[pl.ds(s * rpt, rpt)], acc.at[pl.ds(s * rpt, rpt)])
        pltpu.sync_copy(rowp_hbm.at[wid], sidx)
        plsc.subcore_barrier()

        def body(w, carry):
            pltpu.async_copy(ones, acc.at[sidx.at[w]], ssem, add=True)
            return carry

        lax.fori_loop(0, nwin, body, 0)

        def drain(w, carry):
            pltpu.make_async_copy(ones, acc.at[sidx.at[0]], ssem).wait()
            return carry

        lax.fori_loop(0, nwin, drain, 0)
        plsc.subcore_barrier()
        pltpu.sync_copy(acc.at[pl.ds(s * rpt, rpt)],
                        out_hbm.at[c, pl.ds(s * rpt, rpt)])

    return k(rowp, zeros1)


def _sc_prop_call(u, row, colp, zeros2, np_, d, e):
    epw = e // NW
    ch = 80
    nwin = epw // ch
    rpt = np_ // NS
    mesh = plsc.VectorSubcoreMesh(core_axis_name="c", subcore_axis_name="s")

    @functools.partial(
        pl.kernel,
        out_type=jax.ShapeDtypeStruct((NC, np_, d), jnp.float32),
        mesh=mesh,
        scratch_types=[
            pltpu.VMEM((nwin, ch), jnp.int32),
            pltpu.VMEM((ch,), jnp.int32),
            pltpu.VMEM((ch,), jnp.int32),
            pltpu.VMEM((ch,), jnp.int32),
            pltpu.VMEM((ch, d), jnp.float32),
            pltpu.VMEM((ch, d), jnp.float32),
            pltpu.VMEM((ch, d), jnp.float32),
            pltpu.VMEM_SHARED((np_, d), jnp.float32),
            pltpu.SemaphoreType.DMA,
            pltpu.SemaphoreType.DMA,
            pltpu.SemaphoreType.DMA,
            pltpu.SemaphoreType.DMA,
            pltpu.SemaphoreType.DMA,
            pltpu.SemaphoreType.DMA,
            pltpu.SemaphoreType.DMA,
            pltpu.SemaphoreType.DMA,
            pltpu.SemaphoreType.DMA,
        ],
    )
    def k(u_hbm, row_hbm, colp_hbm, z_hbm, out_hbm,
          sidx, i0, i1, i2, r0, r1, r2, acc,
          gi0, gi1, gi2, gs0, gs1, gs2, ss0, ss1, ss2):
        c = lax.axis_index("c")
        s = lax.axis_index("s")
        wid = c * NS + s
        ibuf = (i0, i1, i2)
        rows = (r0, r1, r2)
        isems = (gi0, gi1, gi2)
        gsems = (gs0, gs1, gs2)
        ssems = (ss0, ss1, ss2)
        pltpu.sync_copy(z_hbm.at[pl.ds(s * rpt, rpt)], acc.at[pl.ds(s * rpt, rpt)])
        pltpu.sync_copy(colp_hbm.at[wid], sidx)
        base = wid * epw

        def iissue(wi, b):
            off = pl.multiple_of(base + wi * ch, 8)
            pltpu.async_copy(row_hbm.at[pl.ds(off, ch)], ibuf[b], isems[b])

        def iwait(b):
            pltpu.make_async_copy(
                row_hbm.at[pl.ds(0, ch)], ibuf[b], isems[b]).wait()

        def gissue(wi, b):
            pltpu.async_copy(u_hbm.at[ibuf[b]], rows[b], gsems[b])

        def gwait(b):
            pltpu.make_async_copy(
                u_hbm.at[ibuf[b]], rows[b], gsems[b]).wait()

        def swait(b):
            pltpu.make_async_copy(rows[b], acc.at[sidx.at[0]], ssems[b]).wait()

        iissue(0, 0)
        iissue(1, 1)
        plsc.subcore_barrier()
        iwait(0)
        gissue(0, 0)
        iwait(1)
        gissue(1, 1)
        iissue(2, 2)

        def body(g, carry):
            for b in range(3):
                wi = g * 3 + b
                bn = (b + 1) % 3
                bp = (b + 2) % 3

                @pl.when(wi < nwin)
                def _():
                    gwait(b)
                    pltpu.async_copy(rows[b], acc.at[sidx.at[wi]], ssems[b],
                                     add=True)

                    @pl.when(wi >= 1)
                    def _():
                        swait(bp)

                    @pl.when(wi + 3 < nwin)
                    def _():
                        iissue(wi + 3, b)

                    @pl.when(wi + 2 < nwin)
                    def _():
                        iwait(bp)
                        gissue(wi + 2, bp)

            return carry

        lax.fori_loop(0, (nwin + 2) // 3, body, 0)
        swait((nwin - 1) % 3)
        plsc.subcore_barrier()
        pltpu.sync_copy(acc.at[pl.ds(s * rpt, rpt)],
                        out_hbm.at[c, pl.ds(s * rpt, rpt)])

    return k(u, row, colp, zeros2)


def _dis_u_call(degp, x_pad, np_, d):
    def k(degp_ref, x_ref, dis_ref, u_ref):
        dg = degp_ref[0, :] + degp_ref[1, :]
        dis = jnp.where(dg > 0, lax.rsqrt(dg), 0.0)
        dis_b = jnp.broadcast_to(dis[:, None], (np_, d))
        dis_ref[...] = dis_b
        u_ref[...] = dis_b * x_ref[...]

    return pl.pallas_call(
        k,
        out_shape=(
            jax.ShapeDtypeStruct((np_, d), jnp.float32),
            jax.ShapeDtypeStruct((np_, d), jnp.float32),
        ),
    )(degp, x_pad)


def _layer_c_call(h, sp, dis_b, w, np_, d):
    blk = 1024

    def k(h_ref, sp_ref, dis_ref, w_ref, outp_ref, v_ref):
        s1 = sp_ref[0] + sp_ref[1]
        dis = dis_ref[...]
        tx1 = -(dis * s1)
        v_ref[...] = dis * tx1
        outp_ref[...] = (
            jnp.dot(h_ref[...], w_ref[0], preferred_element_type=jnp.float32)
            + jnp.dot(tx1, w_ref[1], preferred_element_type=jnp.float32)
        )

    return pl.pallas_call(
        k,
        grid=(np_ // blk,),
        in_specs=[
            pl.BlockSpec((blk, d), lambda i: (i, 0)),
            pl.BlockSpec((NC, blk, d), lambda i: (0, i, 0)),
            pl.BlockSpec((blk, d), lambda i: (i, 0)),
            pl.BlockSpec((3, d, d), lambda i: (0, 0, 0)),
        ],
        out_specs=(
            pl.BlockSpec((blk, d), lambda i: (i, 0)),
            pl.BlockSpec((blk, d), lambda i: (i, 0)),
        ),
        out_shape=(
            jax.ShapeDtypeStruct((np_, d), jnp.float32),
            jax.ShapeDtypeStruct((np_, d), jnp.float32),
        ),
    )(h, sp, dis_b, w)


def _layer_d_call(h, outp, sp, dis_b, w, b, np_, d, relu):
    blk = 1024

    def k(h_ref, outp_ref, sp_ref, dis_ref, w_ref, b_ref, o_ref, u_ref):
        s2 = sp_ref[0] + sp_ref[1]
        dis = dis_ref[...]
        tx2 = -2.0 * (dis * s2) - h_ref[...]
        o = (
            outp_ref[...]
            + jnp.dot(tx2, w_ref[2], preferred_element_type=jnp.float32)
            + b_ref[...]
        )
        if relu:
            o = jnp.maximum(o, 0.0)
        o_ref[...] = o
        u_ref[...] = dis * o

    return pl.pallas_call(
        k,
        grid=(np_ // blk,),
        in_specs=[
            pl.BlockSpec((blk, d), lambda i: (i, 0)),
            pl.BlockSpec((blk, d), lambda i: (i, 0)),
            pl.BlockSpec((NC, blk, d), lambda i: (0, i, 0)),
            pl.BlockSpec((blk, d), lambda i: (i, 0)),
            pl.BlockSpec((3, d, d), lambda i: (0, 0, 0)),
            pl.BlockSpec((1, d), lambda i: (0, 0)),
        ],
        out_specs=(
            pl.BlockSpec((blk, d), lambda i: (i, 0)),
            pl.BlockSpec((blk, d), lambda i: (i, 0)),
        ),
        out_shape=(
            jax.ShapeDtypeStruct((np_, d), jnp.float32),
            jax.ShapeDtypeStruct((np_, d), jnp.float32),
        ),
    )(h, outp, sp, dis_b, w, b)


def kernel(x, edge_index, W1, b1, W2, b2, W3, b3):
    n, d = x.shape
    e = edge_index.shape[1]
    np_ = ((n + 1 + NW * 8 - 1) // (NW * 8)) * (NW * 8)

    epw = e // NW
    ch = 80
    nwin = epw // ch
    rc = _prep_call(edge_index, n, e)
    rowp = rc[0].reshape(NW, nwin, ch)
    colp = rc[1].reshape(NW, nwin, ch)
    row = edge_index[0]

    zeros1 = jnp.zeros((np_,), jnp.float32)
    zeros2 = jnp.zeros((np_, d), jnp.float32)
    x_pad = jnp.pad(x, ((0, np_ - n), (0, 0)))

    degp = _deg_call(rowp, zeros1, np_, e)
    dis_b, u = _dis_u_call(degp, x_pad, np_, d)

    h = x_pad
    for w, b, relu in ((W1, b1, True), (W2, b2, True), (W3, b3, False)):
        b2d = b.reshape(1, d)
        sp1 = _sc_prop_call(u, row, colp, zeros2, np_, d, e)
        outp, v = _layer_c_call(h, sp1, dis_b, w, np_, d)
        sp2 = _sc_prop_call(v, row, colp, zeros2, np_, d, e)
        h, u = _layer_d_call(h, outp, sp2, dis_b, w, b2d, np_, d, relu)

    return h[:n]


# speedup vs baseline: 18.1998x; 1.0259x over previous
import functools

import jax
import jax.numpy as jnp
from jax import lax
from jax.experimental import pallas as pl
from jax.experimental.pallas import tpu as pltpu
from jax.experimental.pallas import tpu_sc as plsc

NC = 2
NS = 16
NW = NC * NS


def _prep_call(edge_index, n, e):
    eb = 32000

    def k(e_ref, o_ref):
        r = e_ref[0, :]
        c = e_ref[1, :]
        self_m = r == c
        o_ref[0, :] = jnp.where(self_m, n, r)
        o_ref[1, :] = jnp.where(self_m, n, c)

    return pl.pallas_call(
        k,
        grid=(e // eb,),
        in_specs=[pl.BlockSpec((2, eb), lambda i: (0, i))],
        out_specs=pl.BlockSpec((2, eb), lambda i: (0, i)),
        out_shape=jax.ShapeDtypeStruct((2, e), jnp.int32),
    )(edge_index)


def _deg_call(rowp, zeros1, np_, e):
    epw = e // NW
    ch = 80
    nwin = epw // ch
    rpt = np_ // NS
    mesh = plsc.VectorSubcoreMesh(core_axis_name="c", subcore_axis_name="s")

    @functools.partial(
        pl.kernel,
        out_type=jax.ShapeDtypeStruct((NC, np_), jnp.float32),
        mesh=mesh,
        scratch_types=[
            pltpu.VMEM((nwin, ch), jnp.int32),
            pltpu.VMEM((ch,), jnp.float32),
            pltpu.VMEM_SHARED((np_,), jnp.float32),
            pltpu.SemaphoreType.DMA,
        ],
    )
    def k(rowp_hbm, z_hbm, out_hbm, sidx, ones, acc, ssem):
        c = lax.axis_index("c")
        s = lax.axis_index("s")
        wid = c * NS + s
        for j in range(ch // 16):
            ones[pl.ds(j * 16, 16)] = jnp.ones((16,), jnp.float32)
        pltpu.sync_copy(z_hbm.at[pl.ds(s * rpt, rpt)], acc.at[pl.ds(s * rpt, rpt)])
        pltpu.sync_copy(rowp_hbm.at[wid], sidx)
        plsc.subcore_barrier()

        def body(w, carry):
            pltpu.async_copy(ones, acc.at[sidx.at[w]], ssem, add=True)
            return carry

        lax.fori_loop(0, nwin, body, 0)

        def drain(w, carry):
            pltpu.make_async_copy(ones, acc.at[sidx.at[0]], ssem).wait()
            return carry

        lax.fori_loop(0, nwin, drain, 0)
        plsc.subcore_barrier()
        pltpu.sync_copy(acc.at[pl.ds(s * rpt, rpt)],
                        out_hbm.at[c, pl.ds(s * rpt, rpt)])

    return k(rowp, zeros1)


def _sc_prop_call(u, row, colp, zeros2, np_, d, e):
    epw = e // NW
    ch = 40
    nb = 8
    nwin = epw // ch
    rpt = np_ // NS
    mesh = plsc.VectorSubcoreMesh(core_axis_name="c", subcore_axis_name="s")

    @functools.partial(
        pl.kernel,
        out_type=jax.ShapeDtypeStruct((NC, np_, d), jnp.float32),
        mesh=mesh,
        scratch_types=(
            [pltpu.VMEM((ch,), jnp.int32) for _ in range(2 * nb)]
            + [pltpu.VMEM((ch, d), jnp.float32) for _ in range(nb)]
            + [pltpu.VMEM_SHARED((np_, d), jnp.float32)]
            + [pltpu.SemaphoreType.DMA for _ in range(3 * nb)]
        ),
    )
    def k(u_hbm, row_hbm, colp_hbm, z_hbm, out_hbm, *rest):
        ibuf = rest[:nb]
        sbuf = rest[nb:2 * nb]
        rows = rest[2 * nb:3 * nb]
        acc = rest[3 * nb]
        isems = rest[3 * nb + 1:4 * nb + 1]
        gsems = rest[4 * nb + 1:5 * nb + 1]
        ssems = rest[5 * nb + 1:6 * nb + 1]
        c = lax.axis_index("c")
        s = lax.axis_index("s")
        wid = c * NS + s
        pltpu.sync_copy(z_hbm.at[pl.ds(s * rpt, rpt)], acc.at[pl.ds(s * rpt, rpt)])
        base = wid * epw

        def pissue(wi, b):
            off = pl.multiple_of(base + wi * ch, 8)
            pltpu.async_copy(row_hbm.at[pl.ds(off, ch)], ibuf[b], isems[b])
            pltpu.async_copy(colp_hbm.at[pl.ds(off, ch)], sbuf[b], isems[b])

        def pwait(b):
            pltpu.make_async_copy(
                row_hbm.at[pl.ds(0, ch)], ibuf[b], isems[b]).wait()
            pltpu.make_async_copy(
                colp_hbm.at[pl.ds(0, ch)], sbuf[b], isems[b]).wait()

        def gissue(b):
            pltpu.async_copy(u_hbm.at[ibuf[b]], rows[b], gsems[b])

        def gwait(b):
            pltpu.make_async_copy(
                u_hbm.at[ibuf[b]], rows[b], gsems[b]).wait()

        def swait(b):
            pltpu.make_async_copy(rows[b], acc.at[sbuf[b]], ssems[b]).wait()

        for j in range(nb - 1):
            pissue(j, j)
        plsc.subcore_barrier()
        for j in range(nb - 2):
            pwait(j)
            gissue(j)

        def body(g, carry):
            for b in range(nb):
                wi = g * nb + b
                bp = (b + nb - 1) % nb
                bpp = (b + nb - 2) % nb

                @pl.when(wi < nwin)
                def _():
                    gwait(b)
                    pltpu.async_copy(rows[b], acc.at[sbuf[b]], ssems[b],
                                     add=True)

                    @pl.when(wi >= 1)
                    def _():
                        swait(bp)

                    @pl.when(wi + nb - 1 < nwin)
                    def _():
                        pissue(wi + nb - 1, bp)

                    @pl.when(wi + nb - 2 < nwin)
                    def _():
                        pwait(bpp)
                        gissue(bpp)

            return carry

        lax.fori_loop(0, (nwin + nb - 1) // nb, body, 0)
        swait((nwin - 1) % nb)
        plsc.subcore_barrier()
        pltpu.sync_copy(acc.at[pl.ds(s * rpt, rpt)],
                        out_hbm.at[c, pl.ds(s * rpt, rpt)])

    return k(u, row, colp, zeros2)


def _dis_u_call(degp, x_pad, np_, d):
    def k(degp_ref, x_ref, dis_ref, u_ref):
        dg = degp_ref[0, :] + degp_ref[1, :]
        dis = jnp.where(dg > 0, lax.rsqrt(dg), 0.0)
        dis_b = jnp.broadcast_to(dis[:, None], (np_, d))
        dis_ref[...] = dis_b
        u_ref[...] = dis_b * x_ref[...]

    return pl.pallas_call(
        k,
        out_shape=(
            jax.ShapeDtypeStruct((np_, d), jnp.float32),
            jax.ShapeDtypeStruct((np_, d), jnp.float32),
        ),
    )(degp, x_pad)


def _layer_c_call(h, sp, dis_b, w, np_, d):
    blk = 1024

    def k(h_ref, sp_ref, dis_ref, w_ref, outp_ref, v_ref):
        s1 = sp_ref[0] + sp_ref[1]
        dis = dis_ref[...]
        tx1 = -(dis * s1)
        v_ref[...] = dis * tx1
        outp_ref[...] = (
            jnp.dot(h_ref[...], w_ref[0], preferred_element_type=jnp.float32)
            + jnp.dot(tx1, w_ref[1], preferred_element_type=jnp.float32)
        )

    return pl.pallas_call(
        k,
        grid=(np_ // blk,),
        in_specs=[
            pl.BlockSpec((blk, d), lambda i: (i, 0)),
            pl.BlockSpec((NC, blk, d), lambda i: (0, i, 0)),
            pl.BlockSpec((blk, d), lambda i: (i, 0)),
            pl.BlockSpec((3, d, d), lambda i: (0, 0, 0)),
        ],
        out_specs=(
            pl.BlockSpec((blk, d), lambda i: (i, 0)),
            pl.BlockSpec((blk, d), lambda i: (i, 0)),
        ),
        out_shape=(
            jax.ShapeDtypeStruct((np_, d), jnp.float32),
            jax.ShapeDtypeStruct((np_, d), jnp.float32),
        ),
    )(h, sp, dis_b, w)


def _layer_d_call(h, outp, sp, dis_b, w, b, np_, d, relu):
    blk = 1024

    def k(h_ref, outp_ref, sp_ref, dis_ref, w_ref, b_ref, o_ref, u_ref):
        s2 = sp_ref[0] + sp_ref[1]
        dis = dis_ref[...]
        tx2 = -2.0 * (dis * s2) - h_ref[...]
        o = (
            outp_ref[...]
            + jnp.dot(tx2, w_ref[2], preferred_element_type=jnp.float32)
            + b_ref[...]
        )
        if relu:
            o = jnp.maximum(o, 0.0)
        o_ref[...] = o
        u_ref[...] = dis * o

    return pl.pallas_call(
        k,
        grid=(np_ // blk,),
        in_specs=[
            pl.BlockSpec((blk, d), lambda i: (i, 0)),
            pl.BlockSpec((blk, d), lambda i: (i, 0)),
            pl.BlockSpec((NC, blk, d), lambda i: (0, i, 0)),
            pl.BlockSpec((blk, d), lambda i: (i, 0)),
            pl.BlockSpec((3, d, d), lambda i: (0, 0, 0)),
            pl.BlockSpec((1, d), lambda i: (0, 0)),
        ],
        out_specs=(
            pl.BlockSpec((blk, d), lambda i: (i, 0)),
            pl.BlockSpec((blk, d), lambda i: (i, 0)),
        ),
        out_shape=(
            jax.ShapeDtypeStruct((np_, d), jnp.float32),
            jax.ShapeDtypeStruct((np_, d), jnp.float32),
        ),
    )(h, outp, sp, dis_b, w, b)


def kernel(x, edge_index, W1, b1, W2, b2, W3, b3):
    n, d = x.shape
    e = edge_index.shape[1]
    np_ = ((n + 1 + NW * 8 - 1) // (NW * 8)) * (NW * 8)

    epw = e // NW
    rc = _prep_call(edge_index, n, e)
    rowp = rc[0].reshape(NW, epw // 80, 80)
    colp = rc[1]
    row = edge_index[0]

    zeros1 = jnp.zeros((np_,), jnp.float32)
    zeros2 = jnp.zeros((np_, d), jnp.float32)
    x_pad = jnp.pad(x, ((0, np_ - n), (0, 0)))

    degp = _deg_call(rowp, zeros1, np_, e)
    dis_b, u = _dis_u_call(degp, x_pad, np_, d)

    h = x_pad
    for w, b, relu in ((W1, b1, True), (W2, b2, True), (W3, b3, False)):
        b2d = b.reshape(1, d)
        sp1 = _sc_prop_call(u, row, colp, zeros2, np_, d, e)
        outp, v = _layer_c_call(h, sp1, dis_b, w, np_, d)
        sp2 = _sc_prop_call(v, row, colp, zeros2, np_, d, e)
        h, u = _layer_d_call(h, outp, sp2, dis_b, w, b2d, np_, d, relu)

    return h[:n]
